# baseline (speedup 1.0000x reference)
import functools

import jax
import jax.numpy as jnp
from jax import lax
from jax.experimental import pallas as pl
from jax.experimental.pallas import tpu as pltpu
from jax.experimental.pallas import tpu_sc as plsc

H = 128
NJ = 16
NJ_HALF = 8
B = 16
LANES = 16
NSUB = 16
CORE0_SHARE = 0.7


def _sc_gather_weighted(feats, idx3, w16, n_pad, q0, q1):
    mesh = plsc.VectorSubcoreMesh(core_axis_name="c", subcore_axis_name="s")
    assert q0 % 2 == 0 and q1 % 2 == 0 and q0 >= 4 and q1 >= 4

    @functools.partial(
        pl.kernel,
        mesh=mesh,
        out_type=jax.ShapeDtypeStruct((n_pad, 2 * H), jnp.float32),
        scratch_types=[
            pltpu.VMEM((2, NJ, B), jnp.int32),
            pltpu.VMEM((2, NJ, B, H), jnp.float32),
            pltpu.VMEM((2, B, 2 * H), jnp.float32),
            pltpu.VMEM((NJ, H), jnp.float32),
            pltpu.SemaphoreType.DMA,
            pltpu.SemaphoreType.DMA,
            pltpu.SemaphoreType.DMA,
            pltpu.SemaphoreType.DMA,
            pltpu.SemaphoreType.DMA,
            pltpu.SemaphoreType.DMA,
        ],
    )
    def k(feats_hbm, idx_hbm, w_hbm, out_hbm, idx_v, g_v, out_v, w_v,
          sem_g0, sem_g1, sem_i0, sem_i1, sem_o0, sem_o1):
        sem_g = (sem_g0, sem_g1)
        sem_i = (sem_i0, sem_i1)
        sem_o = (sem_o0, sem_o1)
        cid = lax.axis_index("c")
        sid = lax.axis_index("s")
        nb = jnp.where(cid == 0, q0, q1)
        base = jnp.where(cid == 0, sid * q0, NSUB * q0 + sid * q1)
        pltpu.sync_copy(w_hbm, w_v)

        pltpu.sync_copy(idx_hbm.at[base], idx_v.at[0])
        for j in range(NJ):
            pltpu.async_copy(feats_hbm.at[idx_v.at[0, j]], g_v.at[0, j], sem_g[0])
        pltpu.async_copy(idx_hbm.at[base + 1], idx_v.at[1], sem_i[1])

        def outer(t2, carry):
            for s in range(2):
                t = t2 * 2 + s
                s2 = 1 - s
                for j in range(NJ):
                    pltpu.make_async_copy(
                        feats_hbm.at[idx_v.at[s, j]], g_v.at[s, j], sem_g[s]
                    ).wait()

                @pl.when(t + 1 < nb)
                def _():
                    pltpu.make_async_copy(
                        idx_hbm.at[base + t + 1], idx_v.at[s2], sem_i[s2]
                    ).wait()
                    for j in range(NJ):
                        pltpu.async_copy(
                            feats_hbm.at[idx_v.at[s2, j]], g_v.at[s2, j], sem_g[s2]
                        )

                @pl.when(t + 2 < nb)
                def _():
                    pltpu.async_copy(idx_hbm.at[base + t + 2], idx_v.at[s], sem_i[s])

                @pl.when(t >= 2)
                def _():
                    pltpu.make_async_copy(
                        out_v.at[s], out_hbm.at[pl.ds((base + t - 2) * B, B)],
                        sem_o[s],
                    ).wait()

                for c in range(H // LANES):
                    co = c * LANES
                    w = [w_v[j, pl.ds(co, LANES)] for j in range(NJ)]

                    @plsc.parallel_loop(0, B, unroll=2)
                    def _(r):
                        acc0 = g_v[s, 0, r, pl.ds(co, LANES)] * w[0]
                        for j in range(1, NJ_HALF):
                            acc0 = acc0 + g_v[s, j, r, pl.ds(co, LANES)] * w[j]
                        acc1 = g_v[s, NJ_HALF, r, pl.ds(co, LANES)] * w[NJ_HALF]
                        for j in range(NJ_HALF + 1, NJ):
                            acc1 = acc1 + g_v[s, j, r, pl.ds(co, LANES)] * w[j]
                        out_v[s, r, pl.ds(co, LANES)] = acc0
                        out_v[s, r, pl.ds(co + H, LANES)] = acc1

                pltpu.async_copy(
                    out_v.at[s], out_hbm.at[pl.ds((base + t) * B, B)], sem_o[s]
                )
            return carry

        lax.fori_loop(0, nb // 2, outer, 0)
        for s in range(2):
            pltpu.make_async_copy(
                out_v.at[s], out_hbm.at[pl.ds((base + nb - 2 + s) * B, B)],
                sem_o[s],
            ).wait()

    return k(feats, idx3, w16)


def _mm_relu_body(x_ref, w_ref, b_ref, o_ref):
    o_ref[...] = jnp.maximum(
        jnp.dot(x_ref[...], w_ref[...], preferred_element_type=jnp.float32)
        + b_ref[...],
        0.0,
    )


def _dense_in(x, w, b, bm, m_out):
    m, kdim = x.shape
    h = w.shape[1]
    return pl.pallas_call(
        _mm_relu_body,
        grid=(m_out // bm,),
        in_specs=[
            pl.BlockSpec((bm, kdim), lambda i: (i, 0)),
            pl.BlockSpec((kdim, h), lambda i: (0, 0)),
            pl.BlockSpec((1, h), lambda i: (0, 0)),
        ],
        out_specs=pl.BlockSpec((bm, h), lambda i: (i, 0)),
        out_shape=jax.ShapeDtypeStruct((m_out, h), jnp.float32),
    )(x, w, b.reshape(1, h))


def _combine_body(alpha, beta, g_ref, pre_ref, inf_ref, w_ref, o_ref):
    fout = jnp.maximum(
        jnp.dot(g_ref[...], w_ref[...], preferred_element_type=jnp.float32), 0.0
    )
    o_ref[...] = (1.0 - alpha - beta) * fout + beta * pre_ref[...] + alpha * inf_ref[...]


def _combine(g, pre, inf, w, alpha, beta, bm):
    m = g.shape[0]
    kdim = g.shape[1]
    h = w.shape[1]
    return pl.pallas_call(
        functools.partial(_combine_body, alpha, beta),
        grid=(m // bm,),
        in_specs=[
            pl.BlockSpec((bm, kdim), lambda i: (i, 0)),
            pl.BlockSpec((bm, h), lambda i: (i, 0)),
            pl.BlockSpec((bm, h), lambda i: (i, 0)),
            pl.BlockSpec((kdim, h), lambda i: (0, 0)),
        ],
        out_specs=pl.BlockSpec((bm, h), lambda i: (i, 0)),
        out_shape=jax.ShapeDtypeStruct((m, h), jnp.float32),
    )(g, pre, inf, w)


def _combine_out_body(alpha, beta, g_ref, pre_ref, inf_ref, w_ref, wo_ref, bo_ref, o_ref):
    fout = jnp.maximum(
        jnp.dot(g_ref[...], w_ref[...], preferred_element_type=jnp.float32), 0.0
    )
    feats = (1.0 - alpha - beta) * fout + beta * pre_ref[...] + alpha * inf_ref[...]
    o_ref[...] = jnp.maximum(
        jnp.dot(feats, wo_ref[...], preferred_element_type=jnp.float32) + bo_ref[...],
        0.0,
    )


def _combine_out(g, pre, inf, w, wo, bo, alpha, beta, bm, m_out):
    m = g.shape[0]
    kdim = g.shape[1]
    h = w.shape[1]
    ho = wo.shape[1]
    return pl.pallas_call(
        functools.partial(_combine_out_body, alpha, beta),
        grid=((m_out + bm - 1) // bm,),
        in_specs=[
            pl.BlockSpec((bm, kdim), lambda i: (i, 0)),
            pl.BlockSpec((bm, h), lambda i: (i, 0)),
            pl.BlockSpec((bm, h), lambda i: (i, 0)),
            pl.BlockSpec((kdim, h), lambda i: (0, 0)),
            pl.BlockSpec((h, ho), lambda i: (0, 0)),
            pl.BlockSpec((1, ho), lambda i: (0, 0)),
        ],
        out_specs=pl.BlockSpec((bm, ho), lambda i: (i, 0)),
        out_shape=jax.ShapeDtypeStruct((m_out, ho), jnp.float32),
    )(g, pre, inf, w, wo, bo.reshape(1, ho))


def kernel(input_x, paths, path_types, fc_in_w, fc_in_b, fc_out_w, fc_out_b,
           layer_fc_w, path_w):
    n, in_dim = input_x.shape
    num_paths, _, path_len = paths.shape
    num_layers = layer_fc_w.shape[0]
    num_types = 2
    alpha, beta = 0.1, 0.1

    min_nb = (n + NSUB * B - 1) // (NSUB * B)
    qsum = ((min_nb + 3) // 4) * 4
    q0 = max(4, 2 * int(round(qsum * CORE0_SHARE / 2)))
    q0 = min(q0, qsum - 4)
    q1 = qsum - q0
    nb = NSUB * qsum
    n_pad = nb * B


    pt16 = jnp.repeat(path_types, path_len)
    perm = jnp.argsort(pt16, stable=True)
    idx16 = paths.transpose(0, 2, 1).reshape(NJ, n)[perm]
    idx16 = jnp.pad(idx16, ((0, 0), (0, n_pad - n)))
    idx3 = idx16.reshape(NJ, nb, B).transpose(1, 0, 2)

    t16 = pt16[perm]
    l16 = jnp.tile(jnp.arange(path_len), num_paths)[perm]
    cnt = jnp.sum(
        path_types[None, :] == jnp.arange(num_types, dtype=path_types.dtype)[:, None],
        axis=1,
    ).astype(jnp.float32)
    w16 = path_w[:, t16, 0, l16, :] / cnt[t16][None, :, None]

    bm = max(g for g in (1024, 512, 256, 128) if n_pad % g == 0)
    in_feats = _dense_in(input_x, fc_in_w, fc_in_b, bm, n_pad)
    feats = in_feats
    for i in range(num_layers):
        g = _sc_gather_weighted(feats, idx3, w16[i], n_pad, q0, q1)
        if i + 1 < num_layers:
            feats = _combine(g, feats, in_feats, layer_fc_w[i], alpha, beta, bm)
        else:
            out = _combine_out(g, feats, in_feats, layer_fc_w[i], fc_out_w,
                               fc_out_b, alpha, beta, bm, n)
    return out


# speedup vs baseline: 1.0105x; 1.0105x over previous
import functools

import jax
import jax.numpy as jnp
from jax import lax
from jax.experimental import pallas as pl
from jax.experimental.pallas import tpu as pltpu
from jax.experimental.pallas import tpu_sc as plsc

H = 128
NJ = 16
NJ_HALF = 8
B = 16
LANES = 16
NSUB = 16
CORE0_SHARE = 0.67


def _sc_gather_weighted(feats, idx3, w16, n_pad, q0, q1):
    mesh = plsc.VectorSubcoreMesh(core_axis_name="c", subcore_axis_name="s")
    assert q0 % 2 == 0 and q1 % 2 == 0 and q0 >= 4 and q1 >= 4

    @functools.partial(
        pl.kernel,
        mesh=mesh,
        out_type=jax.ShapeDtypeStruct((n_pad, 2 * H), jnp.float32),
        scratch_types=[
            pltpu.VMEM((2, NJ, B), jnp.int32),
            pltpu.VMEM((2, NJ, B, H), jnp.float32),
            pltpu.VMEM((2, B, 2 * H), jnp.float32),
            pltpu.VMEM((NJ, H), jnp.float32),
            pltpu.SemaphoreType.DMA,
            pltpu.SemaphoreType.DMA,
            pltpu.SemaphoreType.DMA,
            pltpu.SemaphoreType.DMA,
            pltpu.SemaphoreType.DMA,
            pltpu.SemaphoreType.DMA,
        ],
    )
    def k(feats_hbm, idx_hbm, w_hbm, out_hbm, idx_v, g_v, out_v, w_v,
          sem_g0, sem_g1, sem_i0, sem_i1, sem_o0, sem_o1):
        sem_g = (sem_g0, sem_g1)
        sem_i = (sem_i0, sem_i1)
        sem_o = (sem_o0, sem_o1)
        cid = lax.axis_index("c")
        sid = lax.axis_index("s")
        nb = jnp.where(cid == 0, q0, q1)
        base = jnp.where(cid == 0, sid * q0, NSUB * q0 + sid * q1)
        pltpu.sync_copy(w_hbm, w_v)

        pltpu.sync_copy(idx_hbm.at[base], idx_v.at[0])
        for j in range(NJ):
            pltpu.async_copy(feats_hbm.at[idx_v.at[0, j]], g_v.at[0, j], sem_g[0])
        pltpu.async_copy(idx_hbm.at[base + 1], idx_v.at[1], sem_i[1])

        def outer(t2, carry):
            for s in range(2):
                t = t2 * 2 + s
                s2 = 1 - s
                for j in range(NJ):
                    pltpu.make_async_copy(
                        feats_hbm.at[idx_v.at[s, j]], g_v.at[s, j], sem_g[s]
                    ).wait()

                @pl.when(t + 1 < nb)
                def _():
                    pltpu.make_async_copy(
                        idx_hbm.at[base + t + 1], idx_v.at[s2], sem_i[s2]
                    ).wait()
                    for j in range(NJ):
                        pltpu.async_copy(
                            feats_hbm.at[idx_v.at[s2, j]], g_v.at[s2, j], sem_g[s2]
                        )

                @pl.when(t + 2 < nb)
                def _():
                    pltpu.async_copy(idx_hbm.at[base + t + 2], idx_v.at[s], sem_i[s])

                @pl.when(t >= 2)
                def _():
                    pltpu.make_async_copy(
                        out_v.at[s], out_hbm.at[pl.ds((base + t - 2) * B, B)],
                        sem_o[s],
                    ).wait()

                for c in range(H // LANES):
                    co = c * LANES
                    w = [w_v[j, pl.ds(co, LANES)] for j in range(NJ)]

                    @plsc.parallel_loop(0, B, unroll=2)
                    def _(r):
                        acc0 = g_v[s, 0, r, pl.ds(co, LANES)] * w[0]
                        for j in range(1, NJ_HALF):
                            acc0 = acc0 + g_v[s, j, r, pl.ds(co, LANES)] * w[j]
                        acc1 = g_v[s, NJ_HALF, r, pl.ds(co, LANES)] * w[NJ_HALF]
                        for j in range(NJ_HALF + 1, NJ):
                            acc1 = acc1 + g_v[s, j, r, pl.ds(co, LANES)] * w[j]
                        out_v[s, r, pl.ds(co, LANES)] = acc0
                        out_v[s, r, pl.ds(co + H, LANES)] = acc1

                pltpu.async_copy(
                    out_v.at[s], out_hbm.at[pl.ds((base + t) * B, B)], sem_o[s]
                )
            return carry

        lax.fori_loop(0, nb // 2, outer, 0)
        for s in range(2):
            pltpu.make_async_copy(
                out_v.at[s], out_hbm.at[pl.ds((base + nb - 2 + s) * B, B)],
                sem_o[s],
            ).wait()

    return k(feats, idx3, w16)


def _mm_relu_body(x_ref, w_ref, b_ref, o_ref):
    o_ref[...] = jnp.maximum(
        jnp.dot(x_ref[...], w_ref[...], preferred_element_type=jnp.float32)
        + b_ref[...],
        0.0,
    )


def _dense_in(x, w, b, bm, m_out):
    m, kdim = x.shape
    h = w.shape[1]
    return pl.pallas_call(
        _mm_relu_body,
        grid=(m_out // bm,),
        in_specs=[
            pl.BlockSpec((bm, kdim), lambda i: (i, 0)),
            pl.BlockSpec((kdim, h), lambda i: (0, 0)),
            pl.BlockSpec((1, h), lambda i: (0, 0)),
        ],
        out_specs=pl.BlockSpec((bm, h), lambda i: (i, 0)),
        out_shape=jax.ShapeDtypeStruct((m_out, h), jnp.float32),
    )(x, w, b.reshape(1, h))


def _combine_body(alpha, beta, g_ref, pre_ref, inf_ref, w_ref, o_ref):
    fout = jnp.maximum(
        jnp.dot(g_ref[...], w_ref[...], preferred_element_type=jnp.float32), 0.0
    )
    o_ref[...] = (1.0 - alpha - beta) * fout + beta * pre_ref[...] + alpha * inf_ref[...]


def _combine(g, pre, inf, w, alpha, beta, bm):
    m = g.shape[0]
    kdim = g.shape[1]
    h = w.shape[1]
    return pl.pallas_call(
        functools.partial(_combine_body, alpha, beta),
        grid=(m // bm,),
        in_specs=[
            pl.BlockSpec((bm, kdim), lambda i: (i, 0)),
            pl.BlockSpec((bm, h), lambda i: (i, 0)),
            pl.BlockSpec((bm, h), lambda i: (i, 0)),
            pl.BlockSpec((kdim, h), lambda i: (0, 0)),
        ],
        out_specs=pl.BlockSpec((bm, h), lambda i: (i, 0)),
        out_shape=jax.ShapeDtypeStruct((m, h), jnp.float32),
    )(g, pre, inf, w)


def _combine_out_body(alpha, beta, g_ref, pre_ref, inf_ref, w_ref, wo_ref, bo_ref, o_ref):
    fout = jnp.maximum(
        jnp.dot(g_ref[...], w_ref[...], preferred_element_type=jnp.float32), 0.0
    )
    feats = (1.0 - alpha - beta) * fout + beta * pre_ref[...] + alpha * inf_ref[...]
    o_ref[...] = jnp.maximum(
        jnp.dot(feats, wo_ref[...], preferred_element_type=jnp.float32) + bo_ref[...],
        0.0,
    )


def _combine_out(g, pre, inf, w, wo, bo, alpha, beta, bm, m_out):
    m = g.shape[0]
    kdim = g.shape[1]
    h = w.shape[1]
    ho = wo.shape[1]
    return pl.pallas_call(
        functools.partial(_combine_out_body, alpha, beta),
        grid=((m_out + bm - 1) // bm,),
        in_specs=[
            pl.BlockSpec((bm, kdim), lambda i: (i, 0)),
            pl.BlockSpec((bm, h), lambda i: (i, 0)),
            pl.BlockSpec((bm, h), lambda i: (i, 0)),
            pl.BlockSpec((kdim, h), lambda i: (0, 0)),
            pl.BlockSpec((h, ho), lambda i: (0, 0)),
            pl.BlockSpec((1, ho), lambda i: (0, 0)),
        ],
        out_specs=pl.BlockSpec((bm, ho), lambda i: (i, 0)),
        out_shape=jax.ShapeDtypeStruct((m_out, ho), jnp.float32),
    )(g, pre, inf, w, wo, bo.reshape(1, ho))


def kernel(input_x, paths, path_types, fc_in_w, fc_in_b, fc_out_w, fc_out_b,
           layer_fc_w, path_w):
    n, in_dim = input_x.shape
    num_paths, _, path_len = paths.shape
    num_layers = layer_fc_w.shape[0]
    num_types = 2
    alpha, beta = 0.1, 0.1

    min_nb = (n + NSUB * B - 1) // (NSUB * B)
    qsum = ((min_nb + 3) // 4) * 4
    q0 = max(4, 2 * int(round(qsum * CORE0_SHARE / 2)))
    q0 = min(q0, qsum - 4)
    q1 = qsum - q0
    nb = NSUB * qsum
    n_pad = nb * B


    pt16 = jnp.repeat(path_types, path_len)
    perm = jnp.argsort(pt16, stable=True)
    idx16 = paths.transpose(0, 2, 1).reshape(NJ, n)[perm]
    idx16 = jnp.pad(idx16, ((0, 0), (0, n_pad - n)))
    idx3 = idx16.reshape(NJ, nb, B).transpose(1, 0, 2)

    t16 = pt16[perm]
    l16 = jnp.tile(jnp.arange(path_len), num_paths)[perm]
    cnt = jnp.sum(
        path_types[None, :] == jnp.arange(num_types, dtype=path_types.dtype)[:, None],
        axis=1,
    ).astype(jnp.float32)
    w16 = path_w[:, t16, 0, l16, :] / cnt[t16][None, :, None]

    bm = max(g for g in (1024, 512, 256, 128) if n_pad % g == 0)
    in_feats = _dense_in(input_x, fc_in_w, fc_in_b, bm, n_pad)
    feats = in_feats
    for i in range(num_layers):
        g = _sc_gather_weighted(feats, idx3, w16[i], n_pad, q0, q1)
        if i + 1 < num_layers:
            feats = _combine(g, feats, in_feats, layer_fc_w[i], alpha, beta, bm)
        else:
            out = _combine_out(g, feats, in_feats, layer_fc_w[i], fc_out_w,
                               fc_out_b, alpha, beta, bm, n)
    return out
